# Initial kernel scaffold; baseline (speedup 1.0000x reference)
import jax
import jax.numpy as jnp
from jax.experimental import pallas as pl


def kernel(x, edge_index, W1, b1, W2, b2):
    raise NotImplementedError("write your pallas kernel here")


# speedup vs baseline: 35.8729x; 35.8729x over previous
import functools

import jax
import jax.numpy as jnp
from jax import lax
from jax.experimental import pallas as pl
from jax.experimental.pallas import tpu as pltpu
from jax.experimental.pallas import tpu_sc as plsc

NC = 2
NS = 16
NW = NC * NS

CH = 80
LANES = 16


def _mesh():
    return plsc.VectorSubcoreMesh(
        core_axis_name="c", subcore_axis_name="s", num_cores=NC, num_subcores=NS
    )


def _agg(src_r, dst_r, g, h, n, with_gather):
    _, nch, ch = src_r.shape
    rows_per = n // NS

    scratch = [
        pltpu.VMEM((nch, ch), jnp.int32),
        pltpu.VMEM((nch, ch), jnp.int32),
        pltpu.VMEM((ch, h), jnp.float32),
        pltpu.VMEM((ch, h), jnp.float32),
        pltpu.VMEM((rows_per, h), jnp.float32),
        pltpu.VMEM_SHARED((n, h), jnp.float32),
        pltpu.SemaphoreType.DMA,
        pltpu.SemaphoreType.DMA,
    ]

    @functools.partial(
        pl.kernel,
        out_type=jax.ShapeDtypeStruct((NC, n, h), jnp.float32),
        mesh=_mesh(),
        scratch_types=scratch,
        compiler_params=pltpu.CompilerParams(use_tc_tiling_on_sc=False),
    )
    def body(src_hbm, dst_hbm, g_hbm, out_hbm,
             src_v, dst_v, rg0_v, rg1_v, z_v, acc, sem0, sem1):
        c = lax.axis_index("c")
        s = lax.axis_index("s")
        wid = s * NC + c
        base = s * rows_per

        zv = jnp.zeros((LANES,), jnp.float32)

        def zloop(i, carry):
            for k in range(h // LANES):
                z_v[i, pl.ds(k * LANES, LANES)] = zv
            return carry

        lax.fori_loop(0, rows_per, zloop, 0)
        pltpu.sync_copy(z_v, acc.at[pl.ds(base, rows_per)])

        if not with_gather:
            ov = jnp.ones((LANES,), jnp.float32)

            def oloop(i, carry):
                for k in range(h // LANES):
                    rg0_v[i, pl.ds(k * LANES, LANES)] = ov
                return carry

            lax.fori_loop(0, ch, oloop, 0)

        pltpu.sync_copy(src_hbm.at[wid], src_v)
        pltpu.sync_copy(dst_hbm.at[wid], dst_v)
        plsc.subcore_barrier()

        if with_gather:
            def step(j2, carry):
                a = 2 * j2
                b = a + 1
                cp0 = pltpu.async_copy(g_hbm.at[src_v.at[a]], rg0_v, sem0)
                cp1 = pltpu.async_copy(g_hbm.at[src_v.at[b]], rg1_v, sem1)
                cp0.wait()
                pltpu.sync_copy(rg0_v, acc.at[dst_v.at[a]], add=True)
                cp1.wait()
                pltpu.sync_copy(rg1_v, acc.at[dst_v.at[b]], add=True)
                return carry

            lax.fori_loop(0, nch // 2, step, 0)
            if nch % 2:
                j = nch - 1
                cp = pltpu.async_copy(g_hbm.at[src_v.at[j]], rg0_v, sem0)
                cp.wait()
                pltpu.sync_copy(rg0_v, acc.at[dst_v.at[j]], add=True)
        else:
            def step(j, carry):
                pltpu.sync_copy(rg0_v, acc.at[dst_v.at[j]], add=True)
                return carry

            lax.fori_loop(0, nch, step, 0)

        plsc.subcore_barrier()
        pltpu.sync_copy(acc.at[pl.ds(base, rows_per)],
                        out_hbm.at[c, pl.ds(base, rows_per)])

    return body(src_r, dst_r, g)


def _tc1(x, W1, degp, bn):
    n, f = x.shape
    h1 = W1.shape[1]
    grid = (n // bn,)

    def body(x_ref, w_ref, degp_ref, g_ref, dis_ref):
        deg = degp_ref[0][:, 0:1] + degp_ref[1][:, 0:1] + 1.0
        dis = lax.rsqrt(deg)
        hm = jnp.dot(x_ref[...], w_ref[...], preferred_element_type=jnp.float32)
        g_ref[...] = hm * dis
        dis_ref[...] = dis

    return pl.pallas_call(
        body,
        grid=grid,
        in_specs=[
            pl.BlockSpec((bn, f), lambda i: (i, 0)),
            pl.BlockSpec((f, h1), lambda i: (0, 0)),
            pl.BlockSpec((NC, bn, 16), lambda i: (0, i, 0)),
        ],
        out_specs=[
            pl.BlockSpec((bn, h1), lambda i: (i, 0)),
            pl.BlockSpec((bn, 1), lambda i: (i, 0)),
        ],
        out_shape=[
            jax.ShapeDtypeStruct((n, h1), jnp.float32),
            jax.ShapeDtypeStruct((n, 1), jnp.float32),
        ],
    )(x, W1, degp)


def _tc2(s1p, g1, dis, b1, W2, bn):
    _, n, h1 = s1p.shape
    h2 = W2.shape[1]
    grid = (n // bn,)

    def body(sp_ref, g1_ref, dis_ref, b1_ref, w2_ref, g2_ref):
        dis = dis_ref[...]
        agg = sp_ref[0] + sp_ref[1] + g1_ref[...]
        hidden = jnp.maximum(dis * agg + b1_ref[...], 0.0)
        hm = jnp.dot(hidden, w2_ref[...], preferred_element_type=jnp.float32)
        g2_ref[...] = hm * dis

    return pl.pallas_call(
        body,
        grid=grid,
        in_specs=[
            pl.BlockSpec((NC, bn, h1), lambda i: (0, i, 0)),
            pl.BlockSpec((bn, h1), lambda i: (i, 0)),
            pl.BlockSpec((bn, 1), lambda i: (i, 0)),
            pl.BlockSpec((1, h1), lambda i: (0, 0)),
            pl.BlockSpec((h1, h2), lambda i: (0, 0)),
        ],
        out_specs=pl.BlockSpec((bn, h2), lambda i: (i, 0)),
        out_shape=jax.ShapeDtypeStruct((n, h2), jnp.float32),
    )(s1p, g1, dis, b1, W2)


def _tc3(s2p, g2, dis, b2, bn):
    _, n, h2 = s2p.shape
    grid = (n // bn,)

    def body(sp_ref, g2_ref, dis_ref, b2_ref, out_ref):
        agg = sp_ref[0] + sp_ref[1] + g2_ref[...]
        out_ref[...] = dis_ref[...] * agg + b2_ref[...]

    return pl.pallas_call(
        body,
        grid=grid,
        in_specs=[
            pl.BlockSpec((NC, bn, h2), lambda i: (0, i, 0)),
            pl.BlockSpec((bn, h2), lambda i: (i, 0)),
            pl.BlockSpec((bn, 1), lambda i: (i, 0)),
            pl.BlockSpec((1, h2), lambda i: (0, 0)),
        ],
        out_specs=pl.BlockSpec((bn, h2), lambda i: (i, 0)),
        out_shape=jax.ShapeDtypeStruct((n, h2), jnp.float32),
    )(s2p, g2, dis, b2)


def kernel(x, edge_index, W1, b1, W2, b2):
    n, f = x.shape
    e = edge_index.shape[1]
    h1 = W1.shape[1]
    h2 = W2.shape[1]

    np_ = 10240
    x_p = jnp.pad(x, ((0, np_ - n), (0, 0)))

    ew = e // NW
    nch = ew // CH
    src16 = edge_index[0].reshape(NW, nch, CH)
    dst16 = edge_index[1].reshape(NW, nch, CH)

    gdummy = jnp.zeros((8, h1), jnp.float32)
    degp = _agg(src16, dst16, gdummy, h1, np_, with_gather=False)
    g1, dis = _tc1(x_p, W1, degp, bn=1280)
    s1p = _agg(src16, dst16, g1, h1, np_, with_gather=True)
    g2 = _tc2(s1p, g1, dis, b1.reshape(1, h1), W2, bn=1280)
    s2p = _agg(src16, dst16, g2, h2, np_, with_gather=True)
    out = _tc3(s2p, g2, dis, b2.reshape(1, h2), bn=1280)
    return out[:n]


# speedup vs baseline: 44.2219x; 1.2327x over previous
import functools

import jax
import jax.numpy as jnp
from jax import lax
from jax.experimental import pallas as pl
from jax.experimental.pallas import tpu as pltpu
from jax.experimental.pallas import tpu_sc as plsc

NC = 2
NS = 16
NW = NC * NS

CH = 80
LANES = 16


def _mesh():
    return plsc.VectorSubcoreMesh(
        core_axis_name="c", subcore_axis_name="s", num_cores=NC, num_subcores=NS
    )


def _agg(src_r, dst_r, g, h, n, with_gather):
    _, nch, ch = src_r.shape
    rows_per = n // NS

    scratch = [
        pltpu.VMEM((nch, ch), jnp.int32),
        pltpu.VMEM((nch, ch), jnp.int32),
        pltpu.VMEM((ch, h), jnp.float32),
        pltpu.VMEM((ch, h), jnp.float32),
        pltpu.VMEM((ch, h), jnp.float32),
        pltpu.VMEM((ch, h), jnp.float32),
        pltpu.VMEM((rows_per, h), jnp.float32),
        pltpu.VMEM_SHARED((n, h), jnp.float32),
        pltpu.SemaphoreType.DMA,
        pltpu.SemaphoreType.DMA,
    ]

    @functools.partial(
        pl.kernel,
        out_type=jax.ShapeDtypeStruct((NC, n, h), jnp.float32),
        mesh=_mesh(),
        scratch_types=scratch,
        compiler_params=pltpu.CompilerParams(use_tc_tiling_on_sc=False),
    )
    def body(src_hbm, dst_hbm, g_hbm, out_hbm,
             src_v, dst_v, ra0_v, ra1_v, rb0_v, rb1_v, z_v, acc, sem0, sem1):
        rg0_v = ra0_v
        c = lax.axis_index("c")
        s = lax.axis_index("s")
        wid = s * NC + c
        base = s * rows_per

        zv = jnp.zeros((LANES,), jnp.float32)

        def zloop(i, carry):
            for k in range(h // LANES):
                z_v[i, pl.ds(k * LANES, LANES)] = zv
            return carry

        lax.fori_loop(0, rows_per, zloop, 0)
        pltpu.sync_copy(z_v, acc.at[pl.ds(base, rows_per)])

        if not with_gather:
            ov = jnp.ones((LANES,), jnp.float32)

            def oloop(i, carry):
                for k in range(h // LANES):
                    rg0_v[i, pl.ds(k * LANES, LANES)] = ov
                return carry

            lax.fori_loop(0, ch, oloop, 0)

        pltpu.sync_copy(src_hbm.at[wid], src_v)
        pltpu.sync_copy(dst_hbm.at[wid], dst_v)
        plsc.subcore_barrier()

        if with_gather:
            banks = ((sem0, ra0_v, ra1_v), (sem1, rb0_v, rb1_v))
            ngrp = nch // 2
            npairs = (ngrp - 2) // 2

            def issue(gi, bank):
                sem, r0, r1 = bank
                pltpu.async_copy(g_hbm.at[src_v.at[2 * gi]], r0, sem)
                pltpu.async_copy(g_hbm.at[src_v.at[2 * gi + 1]], r1, sem)

            def drain(gi, bank):
                sem, r0, r1 = bank
                pltpu.make_async_copy(g_hbm.at[src_v.at[2 * gi]], r0, sem).wait()
                pltpu.make_async_copy(g_hbm.at[src_v.at[2 * gi + 1]], r1, sem).wait()
                pltpu.sync_copy(r0, acc.at[dst_v.at[2 * gi]], add=True)
                pltpu.sync_copy(r1, acc.at[dst_v.at[2 * gi + 1]], add=True)

            issue(0, banks[0])
            issue(1, banks[1])

            def step(t, carry):
                ga = 2 * t
                drain(ga, banks[0])
                issue(ga + 2, banks[0])
                drain(ga + 1, banks[1])
                issue(ga + 3, banks[1])
                return carry

            lax.fori_loop(0, npairs, step, 0)
            drain(2 * npairs, banks[0])
            drain(2 * npairs + 1, banks[1])
            for j in range(2 * ngrp, nch):
                cp = pltpu.async_copy(g_hbm.at[src_v.at[j]], ra0_v, sem0)
                cp.wait()
                pltpu.sync_copy(ra0_v, acc.at[dst_v.at[j]], add=True)
        else:
            def step(j, carry):
                pltpu.sync_copy(rg0_v, acc.at[dst_v.at[j]], add=True)
                return carry

            lax.fori_loop(0, nch, step, 0)

        plsc.subcore_barrier()
        pltpu.sync_copy(acc.at[pl.ds(base, rows_per)],
                        out_hbm.at[c, pl.ds(base, rows_per)])

    return body(src_r, dst_r, g)


def _tc1(x, W1, degp, bn):
    n, f = x.shape
    h1 = W1.shape[1]
    grid = (n // bn,)

    def body(x_ref, w_ref, degp_ref, g_ref, dis_ref):
        deg = degp_ref[0][:, 0:1] + degp_ref[1][:, 0:1] + 1.0
        dis = lax.rsqrt(deg)
        hm = jnp.dot(x_ref[...], w_ref[...], preferred_element_type=jnp.float32)
        g_ref[...] = hm * dis
        dis_ref[...] = dis

    return pl.pallas_call(
        body,
        grid=grid,
        in_specs=[
            pl.BlockSpec((bn, f), lambda i: (i, 0)),
            pl.BlockSpec((f, h1), lambda i: (0, 0)),
            pl.BlockSpec((NC, bn, 16), lambda i: (0, i, 0)),
        ],
        out_specs=[
            pl.BlockSpec((bn, h1), lambda i: (i, 0)),
            pl.BlockSpec((bn, 1), lambda i: (i, 0)),
        ],
        out_shape=[
            jax.ShapeDtypeStruct((n, h1), jnp.float32),
            jax.ShapeDtypeStruct((n, 1), jnp.float32),
        ],
    )(x, W1, degp)


def _tc2(s1p, g1, dis, b1, W2, bn):
    _, n, h1 = s1p.shape
    h2 = W2.shape[1]
    grid = (n // bn,)

    def body(sp_ref, g1_ref, dis_ref, b1_ref, w2_ref, g2_ref):
        dis = dis_ref[...]
        agg = sp_ref[0] + sp_ref[1] + g1_ref[...]
        hidden = jnp.maximum(dis * agg + b1_ref[...], 0.0)
        hm = jnp.dot(hidden, w2_ref[...], preferred_element_type=jnp.float32)
        g2_ref[...] = hm * dis

    return pl.pallas_call(
        body,
        grid=grid,
        in_specs=[
            pl.BlockSpec((NC, bn, h1), lambda i: (0, i, 0)),
            pl.BlockSpec((bn, h1), lambda i: (i, 0)),
            pl.BlockSpec((bn, 1), lambda i: (i, 0)),
            pl.BlockSpec((1, h1), lambda i: (0, 0)),
            pl.BlockSpec((h1, h2), lambda i: (0, 0)),
        ],
        out_specs=pl.BlockSpec((bn, h2), lambda i: (i, 0)),
        out_shape=jax.ShapeDtypeStruct((n, h2), jnp.float32),
    )(s1p, g1, dis, b1, W2)


def _tc3(s2p, g2, dis, b2, bn):
    _, n, h2 = s2p.shape
    grid = (n // bn,)

    def body(sp_ref, g2_ref, dis_ref, b2_ref, out_ref):
        agg = sp_ref[0] + sp_ref[1] + g2_ref[...]
        out_ref[...] = dis_ref[...] * agg + b2_ref[...]

    return pl.pallas_call(
        body,
        grid=grid,
        in_specs=[
            pl.BlockSpec((NC, bn, h2), lambda i: (0, i, 0)),
            pl.BlockSpec((bn, h2), lambda i: (i, 0)),
            pl.BlockSpec((bn, 1), lambda i: (i, 0)),
            pl.BlockSpec((1, h2), lambda i: (0, 0)),
        ],
        out_specs=pl.BlockSpec((bn, h2), lambda i: (i, 0)),
        out_shape=jax.ShapeDtypeStruct((n, h2), jnp.float32),
    )(s2p, g2, dis, b2)


def kernel(x, edge_index, W1, b1, W2, b2):
    n, f = x.shape
    e = edge_index.shape[1]
    h1 = W1.shape[1]
    h2 = W2.shape[1]

    np_ = 10240
    x_p = jnp.pad(x, ((0, np_ - n), (0, 0)))

    ew = e // NW
    nch = ew // CH
    src16 = edge_index[0].reshape(NW, nch, CH)
    dst16 = edge_index[1].reshape(NW, nch, CH)

    gdummy = jnp.zeros((8, h1), jnp.float32)
    degp = _agg(src16, dst16, gdummy, h1, np_, with_gather=False)
    g1, dis = _tc1(x_p, W1, degp, bn=1280)
    s1p = _agg(src16, dst16, g1, h1, np_, with_gather=True)
    g2 = _tc2(s1p, g1, dis, b1.reshape(1, h1), W2, bn=1280)
    s2p = _agg(src16, dst16, g2, h2, np_, with_gather=True)
    out = _tc3(s2p, g2, dis, b2.reshape(1, h2), bn=1280)
    return out[:n]


# speedup vs baseline: 46.4581x; 1.0506x over previous
import functools

import jax
import jax.numpy as jnp
from jax import lax
from jax.experimental import pallas as pl
from jax.experimental.pallas import tpu as pltpu
from jax.experimental.pallas import tpu_sc as plsc

NC = 2
NS = 16
NW = NC * NS

CH = 128
CT = 16
LANES = 16


def _mesh():
    return plsc.VectorSubcoreMesh(
        core_axis_name="c", subcore_axis_name="s", num_cores=NC, num_subcores=NS
    )


def _agg(src_a, dst_a, src_b, dst_b, g, h, n, with_gather):
    _, nch, _ = src_a.shape
    rows_per = n // NS

    scratch = [
        pltpu.VMEM((nch, CH), jnp.int32),
        pltpu.VMEM((nch, CH), jnp.int32),
        pltpu.VMEM((CT,), jnp.int32),
        pltpu.VMEM((CT,), jnp.int32),
        pltpu.VMEM((CH, h), jnp.float32),
        pltpu.VMEM((CH, h), jnp.float32),
        pltpu.VMEM((CH, h), jnp.float32),
        pltpu.VMEM((CH, h), jnp.float32),
        pltpu.VMEM((CT, h), jnp.float32),
        pltpu.VMEM((rows_per, h), jnp.float32),
        pltpu.VMEM_SHARED((n, h), jnp.float32),
        pltpu.SemaphoreType.DMA,
        pltpu.SemaphoreType.DMA,
    ]

    @functools.partial(
        pl.kernel,
        out_type=jax.ShapeDtypeStruct((NC, n, h), jnp.float32),
        mesh=_mesh(),
        scratch_types=scratch,
        compiler_params=pltpu.CompilerParams(use_tc_tiling_on_sc=False),
    )
    def body(srca_hbm, dsta_hbm, srcb_hbm, dstb_hbm, g_hbm, out_hbm,
             src_v, dst_v, srct_v, dstt_v,
             ra0_v, ra1_v, rb0_v, rb1_v, rt_v, z_v, acc, sem0, sem1):
        c = lax.axis_index("c")
        s = lax.axis_index("s")
        wid = s * NC + c
        base = s * rows_per

        zv = jnp.zeros((LANES,), jnp.float32)

        def zloop(i, carry):
            for k in range(h // LANES):
                z_v[i, pl.ds(k * LANES, LANES)] = zv
            return carry

        lax.fori_loop(0, rows_per, zloop, 0)
        pltpu.sync_copy(z_v, acc.at[pl.ds(base, rows_per)])

        if not with_gather:
            ov = jnp.ones((LANES,), jnp.float32)

            def oloop(i, carry):
                for k in range(h // LANES):
                    ra0_v[i, pl.ds(k * LANES, LANES)] = ov
                return carry

            lax.fori_loop(0, CH, oloop, 0)

            def otail(i, carry):
                for k in range(h // LANES):
                    rt_v[i, pl.ds(k * LANES, LANES)] = ov
                return carry

            lax.fori_loop(0, CT, otail, 0)

        pltpu.sync_copy(srca_hbm.at[wid], src_v)
        pltpu.sync_copy(dsta_hbm.at[wid], dst_v)
        pltpu.sync_copy(srcb_hbm.at[wid], srct_v)
        pltpu.sync_copy(dstb_hbm.at[wid], dstt_v)
        plsc.subcore_barrier()

        if with_gather:
            banks = ((sem0, ra0_v, ra1_v), (sem1, rb0_v, rb1_v))
            ngrp = nch // 2
            npairs = max((ngrp - 2) // 2, 0)

            def issue(gi, bank):
                sem, r0, r1 = bank
                pltpu.async_copy(g_hbm.at[src_v.at[2 * gi]], r0, sem)
                pltpu.async_copy(g_hbm.at[src_v.at[2 * gi + 1]], r1, sem)

            def drain(gi, bank):
                sem, r0, r1 = bank
                pltpu.make_async_copy(g_hbm.at[src_v.at[2 * gi]], r0, sem).wait()
                pltpu.make_async_copy(g_hbm.at[src_v.at[2 * gi + 1]], r1, sem).wait()
                pltpu.sync_copy(r0, acc.at[dst_v.at[2 * gi]], add=True)
                pltpu.sync_copy(r1, acc.at[dst_v.at[2 * gi + 1]], add=True)

            issue(0, banks[0])
            issue(1, banks[1])

            def step(t, carry):
                ga = 2 * t
                drain(ga, banks[0])
                issue(ga + 2, banks[0])
                drain(ga + 1, banks[1])
                issue(ga + 3, banks[1])
                return carry

            lax.fori_loop(0, npairs, step, 0)
            for k, gi in enumerate(range(2 * npairs, ngrp)):
                if k < 2:
                    drain(gi, banks[k])
                else:
                    sem, r0, r1 = banks[0]
                    pltpu.async_copy(g_hbm.at[src_v.at[2 * gi]], r0, sem).wait()
                    pltpu.sync_copy(r0, acc.at[dst_v.at[2 * gi]], add=True)
                    pltpu.async_copy(g_hbm.at[src_v.at[2 * gi + 1]], r1, sem).wait()
                    pltpu.sync_copy(r1, acc.at[dst_v.at[2 * gi + 1]], add=True)
            for j in range(2 * ngrp, nch):
                cp = pltpu.async_copy(g_hbm.at[src_v.at[j]], ra0_v, sem0)
                cp.wait()
                pltpu.sync_copy(ra0_v, acc.at[dst_v.at[j]], add=True)
            pltpu.async_copy(g_hbm.at[srct_v], rt_v, sem0).wait()
            pltpu.sync_copy(rt_v, acc.at[dstt_v], add=True)
        else:
            def step(j, carry):
                pltpu.sync_copy(ra0_v, acc.at[dst_v.at[j]], add=True)
                return carry

            lax.fori_loop(0, nch, step, 0)
            pltpu.sync_copy(rt_v, acc.at[dstt_v], add=True)

        plsc.subcore_barrier()
        pltpu.sync_copy(acc.at[pl.ds(base, rows_per)],
                        out_hbm.at[c, pl.ds(base, rows_per)])

    return body(src_a, dst_a, src_b, dst_b, g)


def _tc1(x, W1, degp, np_, bn):
    n, f = x.shape
    h1 = W1.shape[1]
    grid = (n // bn,)

    def body(x_ref, w_ref, degp_ref, g_ref, dis_ref):
        deg = degp_ref[0][:, 0:1] + degp_ref[1][:, 0:1] + 1.0
        dis = lax.rsqrt(deg)
        hm = jnp.dot(x_ref[...], w_ref[...], preferred_element_type=jnp.float32)
        g_ref[...] = hm * dis
        dis_ref[...] = dis

    return pl.pallas_call(
        body,
        grid=grid,
        in_specs=[
            pl.BlockSpec((bn, f), lambda i: (i, 0)),
            pl.BlockSpec((f, h1), lambda i: (0, 0)),
            pl.BlockSpec((NC, bn, 16), lambda i: (0, i, 0)),
        ],
        out_specs=[
            pl.BlockSpec((bn, h1), lambda i: (i, 0)),
            pl.BlockSpec((bn, 1), lambda i: (i, 0)),
        ],
        out_shape=[
            jax.ShapeDtypeStruct((np_, h1), jnp.float32),
            jax.ShapeDtypeStruct((np_, 1), jnp.float32),
        ],
    )(x, W1, degp)


def _tc2(s1p, g1, dis, b1, W2, bn):
    _, n, h1 = s1p.shape
    h2 = W2.shape[1]
    grid = (n // bn,)

    def body(sp_ref, g1_ref, dis_ref, b1_ref, w2_ref, g2_ref):
        dis = dis_ref[...]
        agg = sp_ref[0] + sp_ref[1] + g1_ref[...]
        hidden = jnp.maximum(dis * agg + b1_ref[...], 0.0)
        hm = jnp.dot(hidden, w2_ref[...], preferred_element_type=jnp.float32)
        g2_ref[...] = hm * dis

    return pl.pallas_call(
        body,
        grid=grid,
        in_specs=[
            pl.BlockSpec((NC, bn, h1), lambda i: (0, i, 0)),
            pl.BlockSpec((bn, h1), lambda i: (i, 0)),
            pl.BlockSpec((bn, 1), lambda i: (i, 0)),
            pl.BlockSpec((1, h1), lambda i: (0, 0)),
            pl.BlockSpec((h1, h2), lambda i: (0, 0)),
        ],
        out_specs=pl.BlockSpec((bn, h2), lambda i: (i, 0)),
        out_shape=jax.ShapeDtypeStruct((n, h2), jnp.float32),
    )(s1p, g1, dis, b1, W2)


def _tc3(s2p, g2, dis, b2, bn):
    _, n, h2 = s2p.shape
    grid = (n // bn,)

    def body(sp_ref, g2_ref, dis_ref, b2_ref, out_ref):
        agg = sp_ref[0] + sp_ref[1] + g2_ref[...]
        out_ref[...] = dis_ref[...] * agg + b2_ref[...]

    return pl.pallas_call(
        body,
        grid=grid,
        in_specs=[
            pl.BlockSpec((NC, bn, h2), lambda i: (0, i, 0)),
            pl.BlockSpec((bn, h2), lambda i: (i, 0)),
            pl.BlockSpec((bn, 1), lambda i: (i, 0)),
            pl.BlockSpec((1, h2), lambda i: (0, 0)),
        ],
        out_specs=pl.BlockSpec((bn, h2), lambda i: (i, 0)),
        out_shape=jax.ShapeDtypeStruct((n, h2), jnp.float32),
    )(s2p, g2, dis, b2)


def kernel(x, edge_index, W1, b1, W2, b2):
    n, f = x.shape
    e = edge_index.shape[1]
    h1 = W1.shape[1]
    h2 = W2.shape[1]

    np_ = 10240

    ew = e // NW
    nch = (ew - CT) // CH
    e0 = edge_index[0].reshape(NW, ew)
    e1 = edge_index[1].reshape(NW, ew)
    src_a = e0[:, : nch * CH].reshape(NW, nch, CH)
    dst_a = e1[:, : nch * CH].reshape(NW, nch, CH)
    src_b = e0[:, nch * CH :]
    dst_b = e1[:, nch * CH :]

    gdummy = jnp.zeros((8, h1), jnp.float32)
    degp = _agg(src_a, dst_a, src_b, dst_b, gdummy, h1, np_, with_gather=False)
    g1, dis = _tc1(x, W1, degp, np_, bn=2000)
    s1p = _agg(src_a, dst_a, src_b, dst_b, g1, h1, np_, with_gather=True)
    g2 = _tc2(s1p, g1, dis, b1.reshape(1, h1), W2, bn=1280)
    s2p = _agg(src_a, dst_a, src_b, dst_b, g2, h2, np_, with_gather=True)
    out = _tc3(s2p, g2, dis, b2.reshape(1, h2), bn=1280)
    return out[:n]
